# Initial kernel scaffold; baseline (speedup 1.0000x reference)
import jax
import jax.numpy as jnp
from jax.experimental import pallas as pl


def kernel(x_paper, x_author, edge_index_writes, edge_index_rev, params):
    raise NotImplementedError("write your pallas kernel here")


# speedup vs baseline: 31.0940x; 31.0940x over previous
import functools
import numpy as np
import jax
import jax.numpy as jnp
from jax import lax
from jax.experimental import pallas as pl
from jax.experimental.pallas import tpu as pltpu
from jax.experimental.pallas import tpu_sc as plsc

N = 50000
C = 128
H = 4
D = 32
L = 2
E = 300000

CH = 128
NTILE = 16
NCHUNK = 148
TPT = CH * NCHUNK
EPAD = TPT * NTILE
GRP = CH // 16

ZB = 200
ROWS_A = 3200
ROWS_LAST = 2000
NP = 51200

BN = 1000


def _mm(a, w, b):
    return jnp.dot(a, w, preferred_element_type=jnp.float32) + b


def _row_spec():
    return pl.BlockSpec((BN, C), lambda i: (i, 0))


def _w_spec():
    return pl.BlockSpec((C, C), lambda i: (0, 0))


def _b_spec():
    return pl.BlockSpec((1, C), lambda i: (0, 0))


def _stage0_body(x, win, bin_, wk, bk, wv, bv, wq, bq, h_o, k_o, v_o, q_o):
    h = jnp.maximum(_mm(x[...], win[...], bin_[...]), 0.0)
    h_o[...] = h
    k_o[...] = _mm(h, wk[...], bk[...])
    v_o[...] = _mm(h, wv[...], bv[...])
    q_o[...] = _mm(h, wq[...], bq[...])


_stage0 = pl.pallas_call(
    _stage0_body,
    grid=(N // BN,),
    in_specs=[_row_spec()] + [_w_spec(), _b_spec()] * 4,
    out_specs=[_row_spec()] * 4,
    out_shape=[jax.ShapeDtypeStruct((N, C), jnp.float32)] * 4,
)


def _agg_spec(h):
    return pl.BlockSpec((1, BN, D), lambda i, h=h: (h, i, 0))


def _post_common(a0, a1, a2, a3, dent, xprev, gam, wa, ba):
    dv = dent[...]
    parts = []
    for idx, a in enumerate((a0, a1, a2, a3)):
        dh = dv[:, idx:idx + 1] + 1e-16
        parts.append(a[...][0] / dh)
    cat = jnp.concatenate(parts, axis=1)
    o = _mm(jax.nn.gelu(cat), wa[...], ba[...])
    return o + gam[...] * xprev[...]


def _post_proj_body(a0, a1, a2, a3, dent, xprev, gam, wa, ba,
                    wk, bk, wv, bv, wq, bq, h_o, k_o, v_o, q_o):
    hnew = _post_common(a0, a1, a2, a3, dent, xprev, gam, wa, ba)
    h_o[...] = hnew
    k_o[...] = _mm(hnew, wk[...], bk[...])
    v_o[...] = _mm(hnew, wv[...], bv[...])
    q_o[...] = _mm(hnew, wq[...], bq[...])


def _post_final_body(a0, a1, a2, a3, dent, xprev, gam, wa, ba, h_o):
    h_o[...] = _post_common(a0, a1, a2, a3, dent, xprev, gam, wa, ba)


_dent_spec = pl.BlockSpec((BN, H), lambda i: (i, 0))

_post_proj = pl.pallas_call(
    _post_proj_body,
    grid=(N // BN,),
    in_specs=[_agg_spec(0), _agg_spec(1), _agg_spec(2), _agg_spec(3),
              _dent_spec, _row_spec(), _b_spec(), _w_spec(), _b_spec()]
             + [_w_spec(), _b_spec()] * 3,
    out_specs=[_row_spec()] * 4,
    out_shape=[jax.ShapeDtypeStruct((N, C), jnp.float32)] * 4,
)

_post_final = pl.pallas_call(
    _post_final_body,
    grid=(N // BN,),
    in_specs=[_agg_spec(0), _agg_spec(1), _agg_spec(2), _agg_spec(3),
              _dent_spec, _row_spec(), _b_spec(), _w_spec(), _b_spec()],
    out_specs=_row_spec(),
    out_shape=jax.ShapeDtypeStruct((N, C), jnp.float32),
)


def _sc_layer_body(ktw, qtw, vtw, ktr, qtr, vtr, srcw, dstw, srcr, dstr,
                   aggp, denp0, denp1, denp2, denp3,
                   agga, dena0, dena1, dena2, dena3,
                   src_c, dst_c, kidx, qidx, kbuf, qbuf, vbuf, mbuf,
                   exbuf, tscr, zrow, zden, agg_s, den_s, sem):
    c = lax.axis_index("c")
    s = lax.axis_index("s")
    h_base = c * 2
    z16 = jnp.zeros((16,), jnp.float32)
    iot = lax.iota(jnp.int32, 16)
    iot16 = iot * 16

    def _zf(r, carry):
        zrow[r, pl.ds(0, 16)] = z16
        zrow[r, pl.ds(16, 16)] = z16
        return carry
    lax.fori_loop(0, ZB, _zf, 0)
    for i in range(ZB // 16):
        zden[pl.ds(i * 16, 16)] = z16

    start = s * ROWS_A
    ncop = jnp.where(s < 15, ROWS_A // ZB, ROWS_LAST // ZB)
    ebase = s * TPT

    for et in range(2):
        if et == 0:
            ktab, qtab, vtab, srcA, dstA, agg_o = (
                ktw, qtw, vtw, srcw, dstw, aggp)
            den_outs = (denp0, denp1, denp2, denp3)
        else:
            ktab, qtab, vtab, srcA, dstA, agg_o = (
                ktr, qtr, vtr, srcr, dstr, agga)
            den_outs = (dena0, dena1, dena2, dena3)
        for j in range(2):
            h = h_base + j

            def _zb(i, carry):
                off = start + i * ZB
                pltpu.sync_copy(zrow, agg_s.at[pl.ds(off, ZB)])
                return carry
            lax.fori_loop(0, ncop, _zb, 0)
            for i in range(ROWS_A // ZB):
                pltpu.sync_copy(zden, den_s.at[pl.ds(start + i * ZB, ZB)])
            plsc.subcore_barrier()

            def _chunk(ci, carry):
                off = ebase + ci * CH
                pltpu.sync_copy(srcA.at[pl.ds(off, CH)], src_c)
                pltpu.sync_copy(dstA.at[pl.ds(off, CH)], dst_c)
                for g in range(GRP):
                    sl = pl.ds(g * 16, 16)
                    kidx[sl] = src_c[sl] * 4 + h
                    qidx[sl] = jnp.minimum(dst_c[sl], N - 1) * 4 + h
                pltpu.async_copy(ktab.at[kidx], kbuf, sem).wait()
                pltpu.async_copy(qtab.at[qidx], qbuf, sem).wait()
                for g in range(GRP):
                    for r in range(16):
                        row = g * 16 + r
                        p0 = kbuf[row, pl.ds(0, 16)] * qbuf[row, pl.ds(0, 16)]
                        p1 = kbuf[row, pl.ds(16, 16)] * qbuf[row, pl.ds(16, 16)]
                        tscr[pl.ds(r * 16, 16)] = p0 + p1
                    alpha = plsc.load_gather(tscr, [iot16])
                    for jj in range(1, 16):
                        alpha = alpha + plsc.load_gather(tscr, [iot16 + jj])
                    exbuf[pl.ds(g * 16, 16)] = jnp.exp(alpha)
                pltpu.async_copy(vtab.at[kidx], vbuf, sem).wait()
                for g in range(GRP):
                    ex16 = exbuf[pl.ds(g * 16, 16)]
                    for r in range(16):
                        row = g * 16 + r
                        b = jnp.full((16,), ex16[r], jnp.float32)
                        mbuf[row, pl.ds(0, 16)] = vbuf[row, pl.ds(0, 16)] * b
                        mbuf[row, pl.ds(16, 16)] = vbuf[row, pl.ds(16, 16)] * b
                pltpu.sync_copy(mbuf, agg_s.at[dst_c], add=True)
                pltpu.sync_copy(exbuf, den_s.at[dst_c], add=True)
                return carry
            lax.fori_loop(0, NCHUNK, _chunk, 0)
            plsc.subcore_barrier()

            @pl.when(s < 15)
            def _():
                pltpu.sync_copy(agg_s.at[pl.ds(start, ROWS_A)],
                                agg_o.at[h, pl.ds(start, ROWS_A)])

            @pl.when(s == 15)
            def _():
                pltpu.sync_copy(agg_s.at[pl.ds(start, ROWS_LAST)],
                                agg_o.at[h, pl.ds(start, ROWS_LAST)])
            for hh in range(H):
                @pl.when(h == hh)
                def _(hh=hh):
                    pltpu.sync_copy(den_s.at[pl.ds(start, ROWS_A)],
                                    den_outs[hh].at[pl.ds(start, ROWS_A)])
            plsc.subcore_barrier()


_sc_layer = pl.kernel(
    _sc_layer_body,
    out_type=[jax.ShapeDtypeStruct((H, N, D), jnp.float32)]
             + [jax.ShapeDtypeStruct((NP,), jnp.float32)] * 4
             + [jax.ShapeDtypeStruct((H, N, D), jnp.float32)]
             + [jax.ShapeDtypeStruct((NP,), jnp.float32)] * 4,
    mesh=plsc.VectorSubcoreMesh(core_axis_name="c", subcore_axis_name="s"),
    compiler_params=pltpu.CompilerParams(needs_layout_passes=False,
                                         use_tc_tiling_on_sc=False),
    scratch_types=[
        pltpu.VMEM((CH,), jnp.int32),
        pltpu.VMEM((CH,), jnp.int32),
        pltpu.VMEM((CH,), jnp.int32),
        pltpu.VMEM((CH,), jnp.int32),
        pltpu.VMEM((CH, D), jnp.float32),
        pltpu.VMEM((CH, D), jnp.float32),
        pltpu.VMEM((CH, D), jnp.float32),
        pltpu.VMEM((CH, D), jnp.float32),
        pltpu.VMEM((CH,), jnp.float32),
        pltpu.VMEM((256,), jnp.float32),
        pltpu.VMEM((ZB, D), jnp.float32),
        pltpu.VMEM((ZB,), jnp.float32),
        pltpu.VMEM_SHARED((N + 16, D), jnp.float32),
        pltpu.VMEM_SHARED((NP,), jnp.float32),
        pltpu.SemaphoreType.DMA,
    ],
)


def _fold_layer(p):
    out = {}
    rel_of_src = {"author": "author__writes__paper",
                  "paper": "paper__rev_writes__author"}
    for t in ("paper", "author"):
        rp = p["rel"][rel_of_src[t]]
        scale = rp["p_rel"] / np.sqrt(D)
        Wk = p["k"][t]["w"].reshape(C, H, D)
        bk = p["k"][t]["b"].reshape(H, D)
        Wkf = jnp.einsum("chd,hde,h->che", Wk, rp["a_rel"], scale).reshape(C, C)
        bkf = jnp.einsum("hd,hde,h->he", bk, rp["a_rel"], scale).reshape(1, C)
        Wv = p["v"][t]["w"].reshape(C, H, D)
        bv = p["v"][t]["b"].reshape(H, D)
        Wvf = jnp.einsum("chd,hde->che", Wv, rp["m_rel"]).reshape(C, C)
        bvf = jnp.einsum("hd,hde->he", bv, rp["m_rel"]).reshape(1, C)
        beta = jax.nn.sigmoid(p["skip"][t])
        out[t] = {
            "wk": Wkf, "bk": bkf, "wv": Wvf, "bv": bvf,
            "wq": p["q"][t]["w"], "bq": p["q"][t]["b"].reshape(1, C),
            "wa": beta * p["a"][t]["w"], "ba": beta * p["a"][t]["b"].reshape(1, C),
            "gam": jnp.broadcast_to(1.0 - beta, (1, C)),
        }
    return out


def _prep_edges(e):
    src = e[0].astype(jnp.int32)
    dst = e[1].astype(jnp.int32)
    order = jnp.argsort(dst)
    src_s = jnp.concatenate([src[order], jnp.zeros((EPAD - E,), jnp.int32)])
    dst_s = jnp.concatenate([dst[order], jnp.full((EPAD - E,), N, jnp.int32)])
    src_f = src_s.reshape(CH, NTILE * NCHUNK).T.reshape(-1)
    dst_f = dst_s.reshape(CH, NTILE * NCHUNK).T.reshape(-1)
    return src_f, dst_f


_DBG_SC_AGG = True
_DBG_SC_DEN = True


def _jax_pass(ktab, qtab, vtab, src, dst):
    aggs, dens = [], []
    for hh in range(H):
        ke = ktab.reshape(N, H, D)[:, hh, :][src]
        qe = qtab.reshape(N, H, D)[:, hh, :][dst]
        ve = vtab.reshape(N, H, D)[:, hh, :][src]
        ex = jnp.exp((ke * qe).sum(-1))
        dens.append(jax.ops.segment_sum(ex, dst, num_segments=N))
        aggs.append(jax.ops.segment_sum(ve * ex[:, None], dst, num_segments=N))
    return jnp.stack(aggs), jnp.stack(dens).T


def kernel(x_paper, x_author, edge_index_writes, edge_index_rev, params):
    srcw, dstw = _prep_edges(edge_index_writes)
    srcr, dstr = _prep_edges(edge_index_rev)

    folds = [_fold_layer(p) for p in params["layers"]]

    x = {"paper": x_paper, "author": x_author}
    h, ktab, vtab, qtab = {}, {}, {}, {}
    for t in ("paper", "author"):
        f0 = folds[0][t]
        pin = params["in"][t]
        h[t], ktab[t], vtab[t], qtab[t] = _stage0(
            x[t], pin["w"], pin["b"].reshape(1, C),
            f0["wk"], f0["bk"], f0["wv"], f0["bv"], f0["wq"], f0["bq"])

    for l in range(L):
        flat = {t: {"k": ktab[t].reshape(H * N, D),
                    "v": vtab[t].reshape(H * N, D),
                    "q": qtab[t].reshape(H * N, D)} for t in ("paper", "author")}
        (aggp, dp0, dp1, dp2, dp3, agga, da0, da1, da2, da3) = _sc_layer(
            flat["author"]["k"], flat["paper"]["q"], flat["author"]["v"],
            flat["paper"]["k"], flat["author"]["q"], flat["paper"]["v"],
            srcw, dstw, srcr, dstr)
        agg = {"paper": aggp, "author": agga}
        dent = {"paper": jnp.stack([dp0[:N], dp1[:N], dp2[:N], dp3[:N]], axis=1),
                "author": jnp.stack([da0[:N], da1[:N], da2[:N], da3[:N]], axis=1)}
        if not (_DBG_SC_AGG and _DBG_SC_DEN):
            ja_p, jd_p = _jax_pass(
                ktab["author"], qtab["paper"], vtab["author"],
                edge_index_writes[0], edge_index_writes[1])
            ja_a, jd_a = _jax_pass(
                ktab["paper"], qtab["author"], vtab["paper"],
                edge_index_rev[0], edge_index_rev[1])
            if not _DBG_SC_AGG:
                agg = {"paper": ja_p, "author": ja_a}
            if not _DBG_SC_DEN:
                dent = {"paper": jd_p, "author": jd_a}
        newh = {}
        if l + 1 < L:
            for t in ("paper", "author"):
                f = folds[l + 1][t]
                fl = folds[l][t]
                newh[t], ktab[t], vtab[t], qtab[t] = _post_proj(
                    agg[t], agg[t], agg[t], agg[t],
                    dent[t], h[t], fl["gam"], fl["wa"], fl["ba"],
                    f["wk"], f["bk"], f["wv"], f["bv"], f["wq"], f["bq"])
        else:
            for t in ("paper", "author"):
                fl = folds[l][t]
                newh[t] = _post_final(
                    agg[t], agg[t], agg[t], agg[t],
                    dent[t], h[t], fl["gam"], fl["wa"], fl["ba"])
        h = newh

    return h["paper"], h["author"]


# speedup vs baseline: 31.1134x; 1.0006x over previous
import functools
import numpy as np
import jax
import jax.numpy as jnp
from jax import lax
from jax.experimental import pallas as pl
from jax.experimental.pallas import tpu as pltpu
from jax.experimental.pallas import tpu_sc as plsc

N = 50000
C = 128
H = 4
D = 32
L = 2
E = 300000

CH = 128
NTILE = 16
NCHUNK = 148
TPT = CH * NCHUNK
EPAD = TPT * NTILE
GRP = CH // 16

ZB = 200
ROWS_A = 3200
ROWS_LAST = 2000
NP = 51200

BN = 1000


def _mm(a, w, b):
    return jnp.dot(a, w, preferred_element_type=jnp.float32) + b


def _row_spec():
    return pl.BlockSpec((BN, C), lambda i: (i, 0))


def _w_spec():
    return pl.BlockSpec((C, C), lambda i: (0, 0))


def _b_spec():
    return pl.BlockSpec((1, C), lambda i: (0, 0))


def _stage0_body(x, win, bin_, wk, bk, wv, bv, wq, bq, h_o, k_o, v_o, q_o):
    h = jnp.maximum(_mm(x[...], win[...], bin_[...]), 0.0)
    h_o[...] = h
    k_o[...] = _mm(h, wk[...], bk[...])
    v_o[...] = _mm(h, wv[...], bv[...])
    q_o[...] = _mm(h, wq[...], bq[...])


_stage0 = pl.pallas_call(
    _stage0_body,
    grid=(N // BN,),
    in_specs=[_row_spec()] + [_w_spec(), _b_spec()] * 4,
    out_specs=[_row_spec()] * 4,
    out_shape=[jax.ShapeDtypeStruct((N, C), jnp.float32)] * 4,
)


def _agg_spec(h):
    return pl.BlockSpec((1, BN, D), lambda i, h=h: (h, i, 0))


def _post_common(a0, a1, a2, a3, dent, xprev, gam, wa, ba):
    dv = dent[...]
    parts = []
    for idx, a in enumerate((a0, a1, a2, a3)):
        dh = dv[:, idx:idx + 1] + 1e-16
        parts.append(a[...][0] / dh)
    cat = jnp.concatenate(parts, axis=1)
    o = _mm(jax.nn.gelu(cat), wa[...], ba[...])
    return o + gam[...] * xprev[...]


def _post_proj_body(a0, a1, a2, a3, dent, xprev, gam, wa, ba,
                    wk, bk, wv, bv, wq, bq, h_o, k_o, v_o, q_o):
    hnew = _post_common(a0, a1, a2, a3, dent, xprev, gam, wa, ba)
    h_o[...] = hnew
    k_o[...] = _mm(hnew, wk[...], bk[...])
    v_o[...] = _mm(hnew, wv[...], bv[...])
    q_o[...] = _mm(hnew, wq[...], bq[...])


def _post_final_body(a0, a1, a2, a3, dent, xprev, gam, wa, ba, h_o):
    h_o[...] = _post_common(a0, a1, a2, a3, dent, xprev, gam, wa, ba)


_dent_spec = pl.BlockSpec((BN, H), lambda i: (i, 0))

_post_proj = pl.pallas_call(
    _post_proj_body,
    grid=(N // BN,),
    in_specs=[_agg_spec(0), _agg_spec(1), _agg_spec(2), _agg_spec(3),
              _dent_spec, _row_spec(), _b_spec(), _w_spec(), _b_spec()]
             + [_w_spec(), _b_spec()] * 3,
    out_specs=[_row_spec()] * 4,
    out_shape=[jax.ShapeDtypeStruct((N, C), jnp.float32)] * 4,
)

_post_final = pl.pallas_call(
    _post_final_body,
    grid=(N // BN,),
    in_specs=[_agg_spec(0), _agg_spec(1), _agg_spec(2), _agg_spec(3),
              _dent_spec, _row_spec(), _b_spec(), _w_spec(), _b_spec()],
    out_specs=_row_spec(),
    out_shape=jax.ShapeDtypeStruct((N, C), jnp.float32),
)


def _sc_layer_body(ktw, qtw, vtw, ktr, qtr, vtr, srcw, dstw, srcr, dstr,
                   aggp, denp0, denp1, denp2, denp3,
                   agga, dena0, dena1, dena2, dena3,
                   src_c, dst_c, kidx, qidx, kbuf, qbuf, vbuf, mbuf,
                   exbuf, tscr, zrow, zden, agg_s, den_s, sem):
    c = lax.axis_index("c")
    s = lax.axis_index("s")
    h_base = c * 2
    z16 = jnp.zeros((16,), jnp.float32)
    iot = lax.iota(jnp.int32, 16)
    iot16 = iot * 16

    def _zf(r, carry):
        zrow[r, pl.ds(0, 16)] = z16
        zrow[r, pl.ds(16, 16)] = z16
        return carry
    lax.fori_loop(0, ZB, _zf, 0)
    for i in range(ZB // 16):
        zden[pl.ds(i * 16, 16)] = z16

    start = s * ROWS_A
    ncop = jnp.where(s < 15, ROWS_A // ZB, ROWS_LAST // ZB)
    ebase = s * TPT

    for et in range(2):
        if et == 0:
            ktab, qtab, vtab, srcA, dstA, agg_o = (
                ktw, qtw, vtw, srcw, dstw, aggp)
            den_outs = (denp0, denp1, denp2, denp3)
        else:
            ktab, qtab, vtab, srcA, dstA, agg_o = (
                ktr, qtr, vtr, srcr, dstr, agga)
            den_outs = (dena0, dena1, dena2, dena3)
        for j in range(2):
            h = h_base + j

            def _zb(i, carry):
                off = start + i * ZB
                pltpu.sync_copy(zrow, agg_s.at[pl.ds(off, ZB)])
                return carry
            lax.fori_loop(0, ncop, _zb, 0)
            for i in range(ROWS_A // ZB):
                pltpu.sync_copy(zden, den_s.at[pl.ds(start + i * ZB, ZB)])
            plsc.subcore_barrier()

            def _chunk(ci, carry):
                off = ebase + ci * CH
                pltpu.sync_copy(srcA.at[pl.ds(off, CH)], src_c)
                pltpu.sync_copy(dstA.at[pl.ds(off, CH)], dst_c)
                for g in range(GRP):
                    sl = pl.ds(g * 16, 16)
                    kidx[sl] = src_c[sl] * 4 + h
                    qidx[sl] = jnp.minimum(dst_c[sl], N - 1) * 4 + h
                pltpu.async_copy(ktab.at[kidx], kbuf, sem).wait()
                pltpu.async_copy(qtab.at[qidx], qbuf, sem).wait()
                for g in range(GRP):
                    for r in range(16):
                        row = g * 16 + r
                        p0 = kbuf[row, pl.ds(0, 16)] * qbuf[row, pl.ds(0, 16)]
                        p1 = kbuf[row, pl.ds(16, 16)] * qbuf[row, pl.ds(16, 16)]
                        tscr[pl.ds(r * 16, 16)] = p0 + p1
                    alpha = plsc.load_gather(tscr, [iot16])
                    for jj in range(1, 16):
                        alpha = alpha + plsc.load_gather(tscr, [iot16 + jj])
                    exbuf[pl.ds(g * 16, 16)] = jnp.exp(alpha)
                pltpu.async_copy(vtab.at[kidx], vbuf, sem).wait()
                for g in range(GRP):
                    ex16 = exbuf[pl.ds(g * 16, 16)]
                    for r in range(16):
                        row = g * 16 + r
                        b = jnp.full((16,), ex16[r], jnp.float32)
                        mbuf[row, pl.ds(0, 16)] = vbuf[row, pl.ds(0, 16)] * b
                        mbuf[row, pl.ds(16, 16)] = vbuf[row, pl.ds(16, 16)] * b
                pltpu.sync_copy(mbuf, agg_s.at[dst_c], add=True)
                pltpu.sync_copy(exbuf, den_s.at[dst_c], add=True)
                return carry
            lax.fori_loop(0, NCHUNK, _chunk, 0)
            plsc.subcore_barrier()

            @pl.when(s < 15)
            def _():
                pltpu.sync_copy(agg_s.at[pl.ds(start, ROWS_A)],
                                agg_o.at[h, pl.ds(start, ROWS_A)])

            @pl.when(s == 15)
            def _():
                pltpu.sync_copy(agg_s.at[pl.ds(start, ROWS_LAST)],
                                agg_o.at[h, pl.ds(start, ROWS_LAST)])
            for hh in range(H):
                @pl.when(h == hh)
                def _(hh=hh):
                    pltpu.sync_copy(den_s.at[pl.ds(start, ROWS_A)],
                                    den_outs[hh].at[pl.ds(start, ROWS_A)])
            plsc.subcore_barrier()


_sc_layer = pl.kernel(
    _sc_layer_body,
    out_type=[jax.ShapeDtypeStruct((H, N, D), jnp.float32)]
             + [jax.ShapeDtypeStruct((NP,), jnp.float32)] * 4
             + [jax.ShapeDtypeStruct((H, N, D), jnp.float32)]
             + [jax.ShapeDtypeStruct((NP,), jnp.float32)] * 4,
    mesh=plsc.VectorSubcoreMesh(core_axis_name="c", subcore_axis_name="s"),
    compiler_params=pltpu.CompilerParams(needs_layout_passes=False,
                                         use_tc_tiling_on_sc=False),
    scratch_types=[
        pltpu.VMEM((CH,), jnp.int32),
        pltpu.VMEM((CH,), jnp.int32),
        pltpu.VMEM((CH,), jnp.int32),
        pltpu.VMEM((CH,), jnp.int32),
        pltpu.VMEM((CH, D), jnp.float32),
        pltpu.VMEM((CH, D), jnp.float32),
        pltpu.VMEM((CH, D), jnp.float32),
        pltpu.VMEM((CH, D), jnp.float32),
        pltpu.VMEM((CH,), jnp.float32),
        pltpu.VMEM((256,), jnp.float32),
        pltpu.VMEM((ZB, D), jnp.float32),
        pltpu.VMEM((ZB,), jnp.float32),
        pltpu.VMEM_SHARED((N + 16, D), jnp.float32),
        pltpu.VMEM_SHARED((NP,), jnp.float32),
        pltpu.SemaphoreType.DMA,
    ],
)


def _fold_layer(p):
    out = {}
    rel_of_src = {"author": "author__writes__paper",
                  "paper": "paper__rev_writes__author"}
    for t in ("paper", "author"):
        rp = p["rel"][rel_of_src[t]]
        scale = rp["p_rel"] / np.sqrt(D)
        Wk = p["k"][t]["w"].reshape(C, H, D)
        bk = p["k"][t]["b"].reshape(H, D)
        Wkf = jnp.einsum("chd,hde,h->che", Wk, rp["a_rel"], scale).reshape(C, C)
        bkf = jnp.einsum("hd,hde,h->he", bk, rp["a_rel"], scale).reshape(1, C)
        Wv = p["v"][t]["w"].reshape(C, H, D)
        bv = p["v"][t]["b"].reshape(H, D)
        Wvf = jnp.einsum("chd,hde->che", Wv, rp["m_rel"]).reshape(C, C)
        bvf = jnp.einsum("hd,hde->he", bv, rp["m_rel"]).reshape(1, C)
        beta = jax.nn.sigmoid(p["skip"][t])
        out[t] = {
            "wk": Wkf, "bk": bkf, "wv": Wvf, "bv": bvf,
            "wq": p["q"][t]["w"], "bq": p["q"][t]["b"].reshape(1, C),
            "wa": beta * p["a"][t]["w"], "ba": beta * p["a"][t]["b"].reshape(1, C),
            "gam": jnp.broadcast_to(1.0 - beta, (1, C)),
        }
    return out


def _prep_edges(e):
    src = e[0].astype(jnp.int32)
    dst = e[1].astype(jnp.int32)
    order = jnp.argsort(dst)
    src_s = jnp.concatenate([src[order], jnp.zeros((EPAD - E,), jnp.int32)])
    dst_s = jnp.concatenate([dst[order], jnp.full((EPAD - E,), N, jnp.int32)])
    src_f = src_s.reshape(CH, NTILE * NCHUNK).T.reshape(-1)
    dst_f = dst_s.reshape(CH, NTILE * NCHUNK).T.reshape(-1)
    return src_f, dst_f


def kernel(x_paper, x_author, edge_index_writes, edge_index_rev, params):
    srcw, dstw = _prep_edges(edge_index_writes)
    srcr, dstr = _prep_edges(edge_index_rev)

    folds = [_fold_layer(p) for p in params["layers"]]

    x = {"paper": x_paper, "author": x_author}
    h, ktab, vtab, qtab = {}, {}, {}, {}
    for t in ("paper", "author"):
        f0 = folds[0][t]
        pin = params["in"][t]
        h[t], ktab[t], vtab[t], qtab[t] = _stage0(
            x[t], pin["w"], pin["b"].reshape(1, C),
            f0["wk"], f0["bk"], f0["wv"], f0["bv"], f0["wq"], f0["bq"])

    for l in range(L):
        flat = {t: {"k": ktab[t].reshape(H * N, D),
                    "v": vtab[t].reshape(H * N, D),
                    "q": qtab[t].reshape(H * N, D)} for t in ("paper", "author")}
        (aggp, dp0, dp1, dp2, dp3, agga, da0, da1, da2, da3) = _sc_layer(
            flat["author"]["k"], flat["paper"]["q"], flat["author"]["v"],
            flat["paper"]["k"], flat["author"]["q"], flat["paper"]["v"],
            srcw, dstw, srcr, dstr)
        agg = {"paper": aggp, "author": agga}
        dent = {"paper": jnp.stack([dp0[:N], dp1[:N], dp2[:N], dp3[:N]], axis=1),
                "author": jnp.stack([da0[:N], da1[:N], da2[:N], da3[:N]], axis=1)}
        newh = {}
        if l + 1 < L:
            for t in ("paper", "author"):
                f = folds[l + 1][t]
                fl = folds[l][t]
                newh[t], ktab[t], vtab[t], qtab[t] = _post_proj(
                    agg[t], agg[t], agg[t], agg[t],
                    dent[t], h[t], fl["gam"], fl["wa"], fl["ba"],
                    f["wk"], f["bk"], f["wv"], f["bv"], f["wq"], f["bq"])
        else:
            for t in ("paper", "author"):
                fl = folds[l][t]
                newh[t] = _post_final(
                    agg[t], agg[t], agg[t], agg[t],
                    dent[t], h[t], fl["gam"], fl["wa"], fl["ba"])
        h = newh

    return h["paper"], h["author"]


# speedup vs baseline: 40.4004x; 1.2985x over previous
import functools
import numpy as np
import jax
import jax.numpy as jnp
from jax import lax
from jax.experimental import pallas as pl
from jax.experimental.pallas import tpu as pltpu
from jax.experimental.pallas import tpu_sc as plsc

N = 50000
C = 128
H = 4
D = 32
L = 2
E = 300000

CH = 128
NTILE = 16
NCHUNK = 148
TPT = CH * NCHUNK
EPAD = TPT * NTILE
GRP = CH // 16

ZB = 200
ROWS_A = 3200
ROWS_LAST = 2000
NP = 51200

BN = 1000


def _mm(a, w, b):
    return jnp.dot(a, w, preferred_element_type=jnp.float32) + b


def _row_spec():
    return pl.BlockSpec((BN, C), lambda i: (i, 0))


def _w_spec():
    return pl.BlockSpec((C, C), lambda i: (0, 0))


def _b_spec():
    return pl.BlockSpec((1, C), lambda i: (0, 0))


def _stage0_body(x, win, bin_, wk, bk, wv, bv, wq, bq, h_o, k_o, v_o, q_o):
    h = jnp.maximum(_mm(x[...], win[...], bin_[...]), 0.0)
    h_o[...] = h
    k_o[...] = _mm(h, wk[...], bk[...])
    v_o[...] = _mm(h, wv[...], bv[...])
    q_o[...] = _mm(h, wq[...], bq[...])


_stage0 = pl.pallas_call(
    _stage0_body,
    grid=(N // BN,),
    in_specs=[_row_spec()] + [_w_spec(), _b_spec()] * 4,
    out_specs=[_row_spec()] * 4,
    out_shape=[jax.ShapeDtypeStruct((N, C), jnp.float32)] * 4,
)


def _agg_spec(h):
    return pl.BlockSpec((1, BN, D), lambda i, h=h: (h, i, 0))


def _post_common(a0, a1, a2, a3, dent, xprev, gam, wa, ba):
    dv = dent[...]
    parts = []
    for idx, a in enumerate((a0, a1, a2, a3)):
        dh = dv[:, idx:idx + 1] + 1e-16
        parts.append(a[...][0] / dh)
    cat = jnp.concatenate(parts, axis=1)
    o = _mm(jax.nn.gelu(cat), wa[...], ba[...])
    return o + gam[...] * xprev[...]


def _post_proj_body(a0, a1, a2, a3, dent, xprev, gam, wa, ba,
                    wk, bk, wv, bv, wq, bq, h_o, k_o, v_o, q_o):
    hnew = _post_common(a0, a1, a2, a3, dent, xprev, gam, wa, ba)
    h_o[...] = hnew
    k_o[...] = _mm(hnew, wk[...], bk[...])
    v_o[...] = _mm(hnew, wv[...], bv[...])
    q_o[...] = _mm(hnew, wq[...], bq[...])


def _post_final_body(a0, a1, a2, a3, dent, xprev, gam, wa, ba, h_o):
    h_o[...] = _post_common(a0, a1, a2, a3, dent, xprev, gam, wa, ba)


_dent_spec = pl.BlockSpec((BN, H), lambda i: (i, 0))

_post_proj = pl.pallas_call(
    _post_proj_body,
    grid=(N // BN,),
    in_specs=[_agg_spec(0), _agg_spec(1), _agg_spec(2), _agg_spec(3),
              _dent_spec, _row_spec(), _b_spec(), _w_spec(), _b_spec()]
             + [_w_spec(), _b_spec()] * 3,
    out_specs=[_row_spec()] * 4,
    out_shape=[jax.ShapeDtypeStruct((N, C), jnp.float32)] * 4,
)

_post_final = pl.pallas_call(
    _post_final_body,
    grid=(N // BN,),
    in_specs=[_agg_spec(0), _agg_spec(1), _agg_spec(2), _agg_spec(3),
              _dent_spec, _row_spec(), _b_spec(), _w_spec(), _b_spec()],
    out_specs=_row_spec(),
    out_shape=jax.ShapeDtypeStruct((N, C), jnp.float32),
)


def _sc_layer_body(ktw, qtw, vtw, ktr, qtr, vtr, srcw, dstw, srcr, dstr,
                   aggp, denp0, denp1, denp2, denp3,
                   agga, dena0, dena1, dena2, dena3,
                   src_c, dst_c, kidx, qidx, kbuf, qbuf, vbuf, mbuf,
                   exbuf, tscr, zrow, zden, agg_s, den_s, sem, sem2, sem3):
    c = lax.axis_index("c")
    s = lax.axis_index("s")
    h_base = c * 2
    z16 = jnp.zeros((16,), jnp.float32)
    iot = lax.iota(jnp.int32, 16)
    iot16 = iot * 16

    def _zf(r, carry):
        zrow[r, pl.ds(0, 16)] = z16
        zrow[r, pl.ds(16, 16)] = z16
        return carry
    lax.fori_loop(0, ZB, _zf, 0)
    for i in range(ZB // 16):
        zden[pl.ds(i * 16, 16)] = z16

    start = s * ROWS_A
    ncop = jnp.where(s < 15, ROWS_A // ZB, ROWS_LAST // ZB)
    ebase = s * TPT

    for et in range(2):
        if et == 0:
            ktab, qtab, vtab, srcA, dstA, agg_o = (
                ktw, qtw, vtw, srcw, dstw, aggp)
            den_outs = (denp0, denp1, denp2, denp3)
        else:
            ktab, qtab, vtab, srcA, dstA, agg_o = (
                ktr, qtr, vtr, srcr, dstr, agga)
            den_outs = (dena0, dena1, dena2, dena3)
        for j in range(2):
            h = h_base + j

            def _zb(i, carry):
                off = start + i * ZB
                pltpu.sync_copy(zrow, agg_s.at[pl.ds(off, ZB)])
                return carry
            lax.fori_loop(0, ncop, _zb, 0)
            for i in range(ROWS_A // ZB):
                pltpu.sync_copy(zden, den_s.at[pl.ds(start + i * ZB, ZB)])
            plsc.subcore_barrier()

            def _chunk(ci, carry):
                off = ebase + ci * CH
                pltpu.sync_copy(srcA.at[pl.ds(off, CH)], src_c)
                pltpu.sync_copy(dstA.at[pl.ds(off, CH)], dst_c)
                for g in range(GRP):
                    sl = pl.ds(g * 16, 16)
                    kidx[sl] = src_c[sl] * 4 + h
                    qidx[sl] = jnp.minimum(dst_c[sl], N - 1) * 4 + h
                dk = pltpu.async_copy(ktab.at[kidx], kbuf, sem)
                dq = pltpu.async_copy(qtab.at[qidx], qbuf, sem2)
                dv = pltpu.async_copy(vtab.at[kidx], vbuf, sem3)
                dk.wait()
                dq.wait()
                for g in range(GRP):
                    for r in range(16):
                        row = g * 16 + r
                        p0 = kbuf[row, pl.ds(0, 16)] * qbuf[row, pl.ds(0, 16)]
                        p1 = kbuf[row, pl.ds(16, 16)] * qbuf[row, pl.ds(16, 16)]
                        tscr[pl.ds(r * 16, 16)] = p0 + p1
                    alpha = plsc.load_gather(tscr, [iot16])
                    for jj in range(1, 16):
                        alpha = alpha + plsc.load_gather(tscr, [iot16 + jj])
                    exbuf[pl.ds(g * 16, 16)] = jnp.exp(alpha)
                dv.wait()
                for g in range(GRP):
                    ex16 = exbuf[pl.ds(g * 16, 16)]
                    for r in range(16):
                        row = g * 16 + r
                        b = jnp.full((16,), ex16[r], jnp.float32)
                        mbuf[row, pl.ds(0, 16)] = vbuf[row, pl.ds(0, 16)] * b
                        mbuf[row, pl.ds(16, 16)] = vbuf[row, pl.ds(16, 16)] * b
                pltpu.sync_copy(mbuf, agg_s.at[dst_c], add=True)
                pltpu.sync_copy(exbuf, den_s.at[dst_c], add=True)
                return carry
            lax.fori_loop(0, NCHUNK, _chunk, 0)
            plsc.subcore_barrier()

            @pl.when(s < 15)
            def _():
                pltpu.sync_copy(agg_s.at[pl.ds(start, ROWS_A)],
                                agg_o.at[h, pl.ds(start, ROWS_A)])

            @pl.when(s == 15)
            def _():
                pltpu.sync_copy(agg_s.at[pl.ds(start, ROWS_LAST)],
                                agg_o.at[h, pl.ds(start, ROWS_LAST)])
            for hh in range(H):
                @pl.when(h == hh)
                def _(hh=hh):
                    pltpu.sync_copy(den_s.at[pl.ds(start, ROWS_A)],
                                    den_outs[hh].at[pl.ds(start, ROWS_A)])
            plsc.subcore_barrier()


_sc_layer = pl.kernel(
    _sc_layer_body,
    out_type=[jax.ShapeDtypeStruct((H, N, D), jnp.float32)]
             + [jax.ShapeDtypeStruct((NP,), jnp.float32)] * 4
             + [jax.ShapeDtypeStruct((H, N, D), jnp.float32)]
             + [jax.ShapeDtypeStruct((NP,), jnp.float32)] * 4,
    mesh=plsc.VectorSubcoreMesh(core_axis_name="c", subcore_axis_name="s"),
    compiler_params=pltpu.CompilerParams(needs_layout_passes=False,
                                         use_tc_tiling_on_sc=False),
    scratch_types=[
        pltpu.VMEM((CH,), jnp.int32),
        pltpu.VMEM((CH,), jnp.int32),
        pltpu.VMEM((CH,), jnp.int32),
        pltpu.VMEM((CH,), jnp.int32),
        pltpu.VMEM((CH, D), jnp.float32),
        pltpu.VMEM((CH, D), jnp.float32),
        pltpu.VMEM((CH, D), jnp.float32),
        pltpu.VMEM((CH, D), jnp.float32),
        pltpu.VMEM((CH,), jnp.float32),
        pltpu.VMEM((256,), jnp.float32),
        pltpu.VMEM((ZB, D), jnp.float32),
        pltpu.VMEM((ZB,), jnp.float32),
        pltpu.VMEM_SHARED((N + 16, D), jnp.float32),
        pltpu.VMEM_SHARED((NP,), jnp.float32),
        pltpu.SemaphoreType.DMA,
        pltpu.SemaphoreType.DMA,
        pltpu.SemaphoreType.DMA,
    ],
)


def _fold_layer(p):
    out = {}
    rel_of_src = {"author": "author__writes__paper",
                  "paper": "paper__rev_writes__author"}
    for t in ("paper", "author"):
        rp = p["rel"][rel_of_src[t]]
        scale = rp["p_rel"] / np.sqrt(D)
        Wk = p["k"][t]["w"].reshape(C, H, D)
        bk = p["k"][t]["b"].reshape(H, D)
        Wkf = jnp.einsum("chd,hde,h->che", Wk, rp["a_rel"], scale).reshape(C, C)
        bkf = jnp.einsum("hd,hde,h->he", bk, rp["a_rel"], scale).reshape(1, C)
        Wv = p["v"][t]["w"].reshape(C, H, D)
        bv = p["v"][t]["b"].reshape(H, D)
        Wvf = jnp.einsum("chd,hde->che", Wv, rp["m_rel"]).reshape(C, C)
        bvf = jnp.einsum("hd,hde->he", bv, rp["m_rel"]).reshape(1, C)
        beta = jax.nn.sigmoid(p["skip"][t])
        out[t] = {
            "wk": Wkf, "bk": bkf, "wv": Wvf, "bv": bvf,
            "wq": p["q"][t]["w"], "bq": p["q"][t]["b"].reshape(1, C),
            "wa": beta * p["a"][t]["w"], "ba": beta * p["a"][t]["b"].reshape(1, C),
            "gam": jnp.broadcast_to(1.0 - beta, (1, C)),
        }
    return out


def _prep_edges(e):
    src = e[0].astype(jnp.int32)
    dst = e[1].astype(jnp.int32)
    order = jnp.argsort(dst)
    src_s = jnp.concatenate([src[order], jnp.zeros((EPAD - E,), jnp.int32)])
    dst_s = jnp.concatenate([dst[order], jnp.full((EPAD - E,), N, jnp.int32)])
    src_f = src_s.reshape(CH, NTILE * NCHUNK).T.reshape(-1)
    dst_f = dst_s.reshape(CH, NTILE * NCHUNK).T.reshape(-1)
    return src_f, dst_f


def kernel(x_paper, x_author, edge_index_writes, edge_index_rev, params):
    srcw, dstw = _prep_edges(edge_index_writes)
    srcr, dstr = _prep_edges(edge_index_rev)

    folds = [_fold_layer(p) for p in params["layers"]]

    x = {"paper": x_paper, "author": x_author}
    h, ktab, vtab, qtab = {}, {}, {}, {}
    for t in ("paper", "author"):
        f0 = folds[0][t]
        pin = params["in"][t]
        h[t], ktab[t], vtab[t], qtab[t] = _stage0(
            x[t], pin["w"], pin["b"].reshape(1, C),
            f0["wk"], f0["bk"], f0["wv"], f0["bv"], f0["wq"], f0["bq"])

    for l in range(L):
        flat = {t: {"k": ktab[t].reshape(H * N, D),
                    "v": vtab[t].reshape(H * N, D),
                    "q": qtab[t].reshape(H * N, D)} for t in ("paper", "author")}
        (aggp, dp0, dp1, dp2, dp3, agga, da0, da1, da2, da3) = _sc_layer(
            flat["author"]["k"], flat["paper"]["q"], flat["author"]["v"],
            flat["paper"]["k"], flat["author"]["q"], flat["paper"]["v"],
            srcw, dstw, srcr, dstr)
        agg = {"paper": aggp, "author": agga}
        dent = {"paper": jnp.stack([dp0[:N], dp1[:N], dp2[:N], dp3[:N]], axis=1),
                "author": jnp.stack([da0[:N], da1[:N], da2[:N], da3[:N]], axis=1)}
        newh = {}
        if l + 1 < L:
            for t in ("paper", "author"):
                f = folds[l + 1][t]
                fl = folds[l][t]
                newh[t], ktab[t], vtab[t], qtab[t] = _post_proj(
                    agg[t], agg[t], agg[t], agg[t],
                    dent[t], h[t], fl["gam"], fl["wa"], fl["ba"],
                    f["wk"], f["bk"], f["wv"], f["bv"], f["wq"], f["bq"])
        else:
            for t in ("paper", "author"):
                fl = folds[l][t]
                newh[t] = _post_final(
                    agg[t], agg[t], agg[t], agg[t],
                    dent[t], h[t], fl["gam"], fl["wa"], fl["ba"])
        h = newh

    return h["paper"], h["author"]


# speedup vs baseline: 45.5452x; 1.1273x over previous
import functools
import numpy as np
import jax
import jax.numpy as jnp
from jax import lax
from jax.experimental import pallas as pl
from jax.experimental.pallas import tpu as pltpu
from jax.experimental.pallas import tpu_sc as plsc

N = 50000
C = 128
H = 4
D = 32
L = 2
E = 300000

CH = 128
NTILE = 16
NCHUNK = 148
TPT = CH * NCHUNK
EPAD = TPT * NTILE
GRP = CH // 16

ZB = 200
ROWS_A = 3200
ROWS_LAST = 2000
NP = 51200

BN = 1000


def _mm(a, w, b):
    return jnp.dot(a, w, preferred_element_type=jnp.float32) + b


def _row_spec():
    return pl.BlockSpec((BN, C), lambda i: (i, 0))


def _w_spec():
    return pl.BlockSpec((C, C), lambda i: (0, 0))


def _b_spec():
    return pl.BlockSpec((1, C), lambda i: (0, 0))


def _stage0_body(x, win, bin_, wk, bk, wv, bv, wq, bq, h_o, k_o, v_o, q_o):
    h = jnp.maximum(_mm(x[...], win[...], bin_[...]), 0.0)
    h_o[...] = h
    k_o[...] = _mm(h, wk[...], bk[...])
    v_o[...] = _mm(h, wv[...], bv[...])
    q_o[...] = _mm(h, wq[...], bq[...])


_stage0 = pl.pallas_call(
    _stage0_body,
    grid=(N // BN,),
    in_specs=[_row_spec()] + [_w_spec(), _b_spec()] * 4,
    out_specs=[_row_spec()] * 4,
    out_shape=[jax.ShapeDtypeStruct((N, C), jnp.float32)] * 4,
)


def _agg_spec(h):
    return pl.BlockSpec((1, BN, D), lambda i, h=h: (h, i, 0))


def _post_common(a0, a1, a2, a3, dent, xprev, gam, wa, ba):
    dv = dent[...]
    parts = []
    for idx, a in enumerate((a0, a1, a2, a3)):
        dh = dv[:, idx:idx + 1] + 1e-16
        parts.append(a[...][0] / dh)
    cat = jnp.concatenate(parts, axis=1)
    o = _mm(jax.nn.gelu(cat), wa[...], ba[...])
    return o + gam[...] * xprev[...]


def _post_proj_body(a0, a1, a2, a3, dent, xprev, gam, wa, ba,
                    wk, bk, wv, bv, wq, bq, h_o, k_o, v_o, q_o):
    hnew = _post_common(a0, a1, a2, a3, dent, xprev, gam, wa, ba)
    h_o[...] = hnew
    k_o[...] = _mm(hnew, wk[...], bk[...])
    v_o[...] = _mm(hnew, wv[...], bv[...])
    q_o[...] = _mm(hnew, wq[...], bq[...])


def _post_final_body(a0, a1, a2, a3, dent, xprev, gam, wa, ba, h_o):
    h_o[...] = _post_common(a0, a1, a2, a3, dent, xprev, gam, wa, ba)


_dent_spec = pl.BlockSpec((BN, H), lambda i: (i, 0))

_post_proj = pl.pallas_call(
    _post_proj_body,
    grid=(N // BN,),
    in_specs=[_agg_spec(0), _agg_spec(1), _agg_spec(2), _agg_spec(3),
              _dent_spec, _row_spec(), _b_spec(), _w_spec(), _b_spec()]
             + [_w_spec(), _b_spec()] * 3,
    out_specs=[_row_spec()] * 4,
    out_shape=[jax.ShapeDtypeStruct((N, C), jnp.float32)] * 4,
)

_post_final = pl.pallas_call(
    _post_final_body,
    grid=(N // BN,),
    in_specs=[_agg_spec(0), _agg_spec(1), _agg_spec(2), _agg_spec(3),
              _dent_spec, _row_spec(), _b_spec(), _w_spec(), _b_spec()],
    out_specs=_row_spec(),
    out_shape=jax.ShapeDtypeStruct((N, C), jnp.float32),
)


def _sc_layer_body(ktw, qtw, vtw, ktr, qtr, vtr, srcw, dstw, srcr, dstr,
                   aggp, denp0, denp1, denp2, denp3,
                   agga, dena0, dena1, dena2, dena3,
                   src_c, dst_c, kidx, qidx, kbuf, qbuf, vbuf, mbuf,
                   exbuf, tscr, zrow, zden, agg_s, den_s, sem, sem2, sem3):
    c = lax.axis_index("c")
    s = lax.axis_index("s")
    h_base = c * 2
    z16 = jnp.zeros((16,), jnp.float32)
    iot = lax.iota(jnp.int32, 16)
    iot16 = iot * 16

    def _zf(r, carry):
        zrow[r, pl.ds(0, 16)] = z16
        zrow[r, pl.ds(16, 16)] = z16
        return carry
    lax.fori_loop(0, ZB, _zf, 0)
    for i in range(ZB // 16):
        zden[pl.ds(i * 16, 16)] = z16

    start = s * ROWS_A
    ncop = jnp.where(s < 15, ROWS_A // ZB, ROWS_LAST // ZB)
    ebase = s * TPT

    for et in range(2):
        if et == 0:
            ktab, qtab, vtab, srcA, dstA, agg_o = (
                ktw, qtw, vtw, srcw, dstw, aggp)
            den_outs = (denp0, denp1, denp2, denp3)
        else:
            ktab, qtab, vtab, srcA, dstA, agg_o = (
                ktr, qtr, vtr, srcr, dstr, agga)
            den_outs = (dena0, dena1, dena2, dena3)
        for j in range(2):
            h = h_base + j

            def _zb(i, carry):
                off = start + i * ZB
                pltpu.sync_copy(zrow, agg_s.at[pl.ds(off, ZB)])
                return carry
            lax.fori_loop(0, ncop, _zb, 0)
            for i in range(ROWS_A // ZB):
                pltpu.sync_copy(zden, den_s.at[pl.ds(start + i * ZB, ZB)])
            plsc.subcore_barrier()

            def _chunk(ci, carry):
                off = ebase + ci * CH
                ds_ = pltpu.async_copy(srcA.at[pl.ds(off, CH)], src_c, sem)
                dd_ = pltpu.async_copy(dstA.at[pl.ds(off, CH)], dst_c, sem2)
                ds_.wait()
                dd_.wait()
                for g in range(GRP):
                    sl = pl.ds(g * 16, 16)
                    kidx[sl] = src_c[sl] * 4 + h
                    qidx[sl] = jnp.minimum(dst_c[sl], N - 1) * 4 + h
                dk = pltpu.async_copy(ktab.at[kidx], kbuf, sem)
                dq = pltpu.async_copy(qtab.at[qidx], qbuf, sem2)
                dv = pltpu.async_copy(vtab.at[kidx], vbuf, sem3)
                dk.wait()
                dq.wait()
                for g in range(GRP):
                    for r in range(16):
                        row = g * 16 + r
                        p0 = kbuf[row, pl.ds(0, 16)] * qbuf[row, pl.ds(0, 16)]
                        p1 = kbuf[row, pl.ds(16, 16)] * qbuf[row, pl.ds(16, 16)]
                        tscr[pl.ds(r * 16, 16)] = p0 + p1
                    alpha = plsc.load_gather(tscr, [iot16])
                    for jj in range(1, 16):
                        alpha = alpha + plsc.load_gather(tscr, [iot16 + jj])
                    exbuf[pl.ds(g * 16, 16)] = jnp.exp(alpha)
                dv.wait()
                for g in range(GRP):
                    ex16 = exbuf[pl.ds(g * 16, 16)]
                    for r in range(16):
                        row = g * 16 + r
                        b = jnp.full((16,), ex16[r], jnp.float32)
                        mbuf[row, pl.ds(0, 16)] = vbuf[row, pl.ds(0, 16)] * b
                        mbuf[row, pl.ds(16, 16)] = vbuf[row, pl.ds(16, 16)] * b
                da_ = pltpu.async_copy(mbuf, agg_s.at[dst_c], sem, add=True)
                de_ = pltpu.async_copy(exbuf, den_s.at[dst_c], sem2, add=True)
                da_.wait()
                de_.wait()
                return carry
            lax.fori_loop(0, NCHUNK, _chunk, 0)
            plsc.subcore_barrier()

            @pl.when(s < 15)
            def _():
                pltpu.sync_copy(agg_s.at[pl.ds(start, ROWS_A)],
                                agg_o.at[h, pl.ds(start, ROWS_A)])

            @pl.when(s == 15)
            def _():
                pltpu.sync_copy(agg_s.at[pl.ds(start, ROWS_LAST)],
                                agg_o.at[h, pl.ds(start, ROWS_LAST)])
            for hh in range(H):
                @pl.when(h == hh)
                def _(hh=hh):
                    pltpu.sync_copy(den_s.at[pl.ds(start, ROWS_A)],
                                    den_outs[hh].at[pl.ds(start, ROWS_A)])
            plsc.subcore_barrier()


_sc_layer = pl.kernel(
    _sc_layer_body,
    out_type=[jax.ShapeDtypeStruct((H, N, D), jnp.float32)]
             + [jax.ShapeDtypeStruct((NP,), jnp.float32)] * 4
             + [jax.ShapeDtypeStruct((H, N, D), jnp.float32)]
             + [jax.ShapeDtypeStruct((NP,), jnp.float32)] * 4,
    mesh=plsc.VectorSubcoreMesh(core_axis_name="c", subcore_axis_name="s"),
    compiler_params=pltpu.CompilerParams(needs_layout_passes=False,
                                         use_tc_tiling_on_sc=False),
    scratch_types=[
        pltpu.VMEM((CH,), jnp.int32),
        pltpu.VMEM((CH,), jnp.int32),
        pltpu.VMEM((CH,), jnp.int32),
        pltpu.VMEM((CH,), jnp.int32),
        pltpu.VMEM((CH, D), jnp.float32),
        pltpu.VMEM((CH, D), jnp.float32),
        pltpu.VMEM((CH, D), jnp.float32),
        pltpu.VMEM((CH, D), jnp.float32),
        pltpu.VMEM((CH,), jnp.float32),
        pltpu.VMEM((256,), jnp.float32),
        pltpu.VMEM((ZB, D), jnp.float32),
        pltpu.VMEM((ZB,), jnp.float32),
        pltpu.VMEM_SHARED((N + 16, D), jnp.float32),
        pltpu.VMEM_SHARED((NP,), jnp.float32),
        pltpu.SemaphoreType.DMA,
        pltpu.SemaphoreType.DMA,
        pltpu.SemaphoreType.DMA,
    ],
)


def _fold_layer(p):
    out = {}
    rel_of_src = {"author": "author__writes__paper",
                  "paper": "paper__rev_writes__author"}
    for t in ("paper", "author"):
        rp = p["rel"][rel_of_src[t]]
        scale = rp["p_rel"] / np.sqrt(D)
        Wk = p["k"][t]["w"].reshape(C, H, D)
        bk = p["k"][t]["b"].reshape(H, D)
        Wkf = jnp.einsum("chd,hde,h->che", Wk, rp["a_rel"], scale).reshape(C, C)
        bkf = jnp.einsum("hd,hde,h->he", bk, rp["a_rel"], scale).reshape(1, C)
        Wv = p["v"][t]["w"].reshape(C, H, D)
        bv = p["v"][t]["b"].reshape(H, D)
        Wvf = jnp.einsum("chd,hde->che", Wv, rp["m_rel"]).reshape(C, C)
        bvf = jnp.einsum("hd,hde->he", bv, rp["m_rel"]).reshape(1, C)
        beta = jax.nn.sigmoid(p["skip"][t])
        out[t] = {
            "wk": Wkf, "bk": bkf, "wv": Wvf, "bv": bvf,
            "wq": p["q"][t]["w"], "bq": p["q"][t]["b"].reshape(1, C),
            "wa": beta * p["a"][t]["w"], "ba": beta * p["a"][t]["b"].reshape(1, C),
            "gam": jnp.broadcast_to(1.0 - beta, (1, C)),
        }
    return out


def _prep_edges(e):
    src = e[0].astype(jnp.int32)
    dst = e[1].astype(jnp.int32)
    order = jnp.argsort(dst)
    src_s = jnp.concatenate([src[order], jnp.zeros((EPAD - E,), jnp.int32)])
    dst_s = jnp.concatenate([dst[order], jnp.full((EPAD - E,), N, jnp.int32)])
    src_f = src_s.reshape(CH, NTILE * NCHUNK).T.reshape(-1)
    dst_f = dst_s.reshape(CH, NTILE * NCHUNK).T.reshape(-1)
    return src_f, dst_f


def kernel(x_paper, x_author, edge_index_writes, edge_index_rev, params):
    srcw, dstw = _prep_edges(edge_index_writes)
    srcr, dstr = _prep_edges(edge_index_rev)

    folds = [_fold_layer(p) for p in params["layers"]]

    x = {"paper": x_paper, "author": x_author}
    h, ktab, vtab, qtab = {}, {}, {}, {}
    for t in ("paper", "author"):
        f0 = folds[0][t]
        pin = params["in"][t]
        h[t], ktab[t], vtab[t], qtab[t] = _stage0(
            x[t], pin["w"], pin["b"].reshape(1, C),
            f0["wk"], f0["bk"], f0["wv"], f0["bv"], f0["wq"], f0["bq"])

    for l in range(L):
        flat = {t: {"k": ktab[t].reshape(H * N, D),
                    "v": vtab[t].reshape(H * N, D),
                    "q": qtab[t].reshape(H * N, D)} for t in ("paper", "author")}
        (aggp, dp0, dp1, dp2, dp3, agga, da0, da1, da2, da3) = _sc_layer(
            flat["author"]["k"], flat["paper"]["q"], flat["author"]["v"],
            flat["paper"]["k"], flat["author"]["q"], flat["paper"]["v"],
            srcw, dstw, srcr, dstr)
        agg = {"paper": aggp, "author": agga}
        dent = {"paper": jnp.stack([dp0[:N], dp1[:N], dp2[:N], dp3[:N]], axis=1),
                "author": jnp.stack([da0[:N], da1[:N], da2[:N], da3[:N]], axis=1)}
        newh = {}
        if l + 1 < L:
            for t in ("paper", "author"):
                f = folds[l + 1][t]
                fl = folds[l][t]
                newh[t], ktab[t], vtab[t], qtab[t] = _post_proj(
                    agg[t], agg[t], agg[t], agg[t],
                    dent[t], h[t], fl["gam"], fl["wa"], fl["ba"],
                    f["wk"], f["bk"], f["wv"], f["bv"], f["wq"], f["bq"])
        else:
            for t in ("paper", "author"):
                fl = folds[l][t]
                newh[t] = _post_final(
                    agg[t], agg[t], agg[t], agg[t],
                    dent[t], h[t], fl["gam"], fl["wa"], fl["ba"])
        h = newh

    return h["paper"], h["author"]
